# Initial kernel scaffold; baseline (speedup 1.0000x reference)
import jax
import jax.numpy as jnp
from jax.experimental import pallas as pl


def kernel(cnn_feature, i_it_ctrs, c_it_ctrs, ind, params):
    raise NotImplementedError("write your pallas kernel here")


# speedup vs baseline: 1.8000x; 1.8000x over previous
import functools

import jax
import jax.numpy as jnp
from jax.experimental import pallas as pl

STATE = 64
FEAT_C = 64
RO = 4.0
ITER = 3
N, P = 1024, 128
NB = 64


def _gcn_body(feat_ref, poly_ref, cpoly_ref,
              w_in, b_in,
              ws0, wn0, b0, ws1, wn1, b1, ws2, wn2, b2, ws3, wn3, b3,
              w_h, b_h, w_out, b_out,
              pred_ref, npoly_ref, ncpoly_ref):
    nb = feat_ref.shape[1]
    feat = feat_ref[...]
    cp = cpoly_ref[...]
    x = jnp.concatenate([feat, cp * RO], axis=0).reshape(FEAT_C + 2, nb * P)
    h = jax.nn.relu(jnp.dot(w_in[...], x, preferred_element_type=jnp.float32)
                    + b_in[...])
    layers = ((ws0, wn0, b0), (ws1, wn1, b1), (ws2, wn2, b2), (ws3, wn3, b3))
    for ws, wn, b in layers:
        h3 = h.reshape(STATE, nb, P)
        prev = jnp.concatenate([h3[:, :, -1:], h3[:, :, :-1]], axis=2)
        nxt = jnp.concatenate([h3[:, :, 1:], h3[:, :, :1]], axis=2)
        nbr = (prev + nxt).reshape(STATE, nb * P)
        h = jax.nn.relu(jnp.dot(ws[...], h, preferred_element_type=jnp.float32)
                        + jnp.dot(wn[...], nbr, preferred_element_type=jnp.float32)
                        + b[...])
    z = jax.nn.relu(jnp.dot(w_h[...], h, preferred_element_type=jnp.float32)
                    + b_h[...])
    off = jnp.dot(w_out[...], z, preferred_element_type=jnp.float32) + b_out[...]
    pred = poly_ref[...] * RO + off.reshape(2, nb, P)
    pred_ref[...] = pred
    npoly = pred * (1.0 / RO)
    npoly_ref[...] = npoly
    ncpoly_ref[...] = npoly - jnp.min(npoly, axis=2, keepdims=True)


def _gcn_stage(feat, poly, cpoly, p):
    grid = (N // NB,)
    data_spec3 = lambda c: pl.BlockSpec((c, NB, P), lambda i: (0, i, 0))
    full = lambda a: pl.BlockSpec(a.shape, lambda i: (0,) * a.ndim)
    weights = [p['W_in'], p['b_in'].reshape(STATE, 1)]
    for l in range(4):
        weights += [p['Ws%d' % l], p['Wn%d' % l], p['b%d' % l].reshape(STATE, 1)]
    weights += [p['W_h'], p['b_h'].reshape(STATE, 1),
                p['W_out'], p['b_out'].reshape(2, 1)]
    out_shape = [jax.ShapeDtypeStruct((2, N, P), jnp.float32)] * 3
    return pl.pallas_call(
        _gcn_body,
        grid=grid,
        in_specs=[data_spec3(FEAT_C), data_spec3(2), data_spec3(2)]
                 + [full(w) for w in weights],
        out_specs=[data_spec3(2)] * 3,
        out_shape=out_shape,
    )(feat, poly, cpoly, *weights)


def _bilinear_gather(fm_rows, base, xs, ys):
    W = 128.0
    x = jnp.clip(xs, 0.0, W - 1.0)
    y = jnp.clip(ys, 0.0, W - 1.0)
    x0 = jnp.floor(x)
    y0 = jnp.floor(y)
    x1 = jnp.minimum(x0 + 1.0, W - 1.0)
    y1 = jnp.minimum(y0 + 1.0, W - 1.0)
    wx = x - x0
    wy = y - y0
    x0i = x0.astype(jnp.int32)
    x1i = x1.astype(jnp.int32)
    y0i = y0.astype(jnp.int32)
    y1i = y1.astype(jnp.int32)
    v00 = jnp.take(fm_rows, base + y0i * 128 + x0i, axis=0)
    v01 = jnp.take(fm_rows, base + y0i * 128 + x1i, axis=0)
    v10 = jnp.take(fm_rows, base + y1i * 128 + x0i, axis=0)
    v11 = jnp.take(fm_rows, base + y1i * 128 + x1i, axis=0)
    out = (v00 * ((1 - wx) * (1 - wy))[:, None] + v01 * (wx * (1 - wy))[:, None]
           + v10 * ((1 - wx) * wy)[:, None] + v11 * (wx * wy)[:, None])
    return out.T.reshape(FEAT_C, N, P)


def kernel(cnn_feature, i_it_ctrs, c_it_ctrs, ind, params):
    B, C, H, W = cnn_feature.shape
    fm_rows = cnn_feature.transpose(0, 2, 3, 1).reshape(B * H * W, C)
    base = jnp.repeat(ind.astype(jnp.int32) * (H * W), P)

    poly = i_it_ctrs.transpose(2, 0, 1)
    cpoly = c_it_ctrs.transpose(2, 0, 1)

    preds = []
    for stage in range(1 + ITER):
        p = params['resgcn'] if stage == 0 else params['resgcn%d' % (stage - 1)]
        xs = poly[0].reshape(N * P)
        ys = poly[1].reshape(N * P)
        feat = _bilinear_gather(fm_rows, base, xs, ys)
        pred, poly, cpoly = _gcn_stage(feat, poly, cpoly, p)
        preds.append(pred)
    return jnp.stack([pr.transpose(1, 2, 0) for pr in preds])
